# Initial kernel scaffold; baseline (speedup 1.0000x reference)
import jax
import jax.numpy as jnp
from jax.experimental import pallas as pl


def kernel(anchor_input_ids, positive_input_ids, negative_input_ids, embedding_table):
    raise NotImplementedError("write your pallas kernel here")


# speedup vs baseline: 2.7441x; 2.7441x over previous
import functools

import jax
import jax.numpy as jnp
import numpy as np
from jax import lax
from jax.experimental import pallas as pl
from jax.experimental.pallas import tpu as pltpu
from jax.experimental.pallas import tpu_sc as plsc

_NUM_EMB = 1_000_000
_DIM = 32
_LANES = 128


def _row_mean_body(x_ref, m_ref, o_ref):
    o_ref[...] = jnp.dot(x_ref[...], m_ref[...],
                         preferred_element_type=jnp.float32)


def _row_means(table):
    fold = _LANES // _DIM
    t = table.reshape(_NUM_EMB // fold, _LANES)
    rows = t.shape[0]
    blk = 10_000
    m = jnp.asarray(np.repeat(np.eye(fold, dtype=np.float32) / _DIM,
                              _DIM, axis=0))
    out = pl.pallas_call(
        _row_mean_body,
        grid=(rows // blk,),
        in_specs=[pl.BlockSpec((blk, _LANES), lambda i: (i, 0)),
                  pl.BlockSpec((_LANES, fold), lambda i: (0, 0))],
        out_specs=pl.BlockSpec((blk, fold), lambda i: (i, 0)),
        out_shape=jax.ShapeDtypeStruct((rows, fold), jnp.float32),
    )(t, m)
    return out.reshape(_NUM_EMB)


def _gather_means(means, idx2d):
    info = plsc.get_sparse_core_info()
    nw = info.num_cores * info.num_subcores
    rows = idx2d.shape[0]
    rpw = rows // nw
    k = 10
    idx3d = idx2d.reshape(nw, rpw, _LANES)
    mesh = plsc.VectorSubcoreMesh(core_axis_name="c", subcore_axis_name="s")

    @functools.partial(
        pl.kernel, mesh=mesh,
        out_type=jax.ShapeDtypeStruct((nw, rpw, _LANES), jnp.float32),
        scratch_types=[
            pltpu.VMEM((rpw, _LANES), jnp.int32),
            pltpu.VMEM((rpw, _LANES), jnp.float32),
            pltpu.SemaphoreType.DMA,
        ],
    )
    def gather_kernel(means_hbm, idx_hbm, out_hbm, idx_v, vals_v, sem):
        wid = lax.axis_index("s") * info.num_cores + lax.axis_index("c")
        pltpu.sync_copy(idx_hbm.at[wid], idx_v)

        def outer(j0, carry):
            descs = [
                pltpu.async_copy(means_hbm.at[idx_v.at[j0 * k + b]],
                                 vals_v.at[j0 * k + b], sem)
                for b in range(k)
            ]
            for d in descs:
                d.wait()
            return carry

        lax.fori_loop(0, rpw // k, outer, 0)
        pltpu.sync_copy(vals_v, out_hbm.at[wid])

    return gather_kernel(means, idx3d).reshape(rows, _LANES)


def _norm_body(x_ref, o_ref):
    x = x_ref[...]
    ss = jnp.sum(x * x, axis=1, keepdims=True)
    o_ref[...] = x / jnp.sqrt(ss)


def _normalize(vals):
    seqs, seq_len = vals.shape
    blk = 4096
    return pl.pallas_call(
        _norm_body,
        grid=(seqs // blk,),
        in_specs=[pl.BlockSpec((blk, seq_len), lambda i: (i, 0))],
        out_specs=pl.BlockSpec((blk, seq_len), lambda i: (i, 0)),
        out_shape=jax.ShapeDtypeStruct((seqs, seq_len), jnp.float32),
    )(vals)


def kernel(anchor_input_ids, positive_input_ids, negative_input_ids,
           embedding_table):
    batch, seq = anchor_input_ids.shape
    num_neg = negative_input_ids.shape[1]

    means = _row_means(embedding_table)
    ids = jnp.concatenate([
        anchor_input_ids.reshape(-1),
        positive_input_ids.reshape(-1),
        negative_input_ids.reshape(-1),
    ]).astype(jnp.int32)
    vals = _gather_means(means, ids.reshape(-1, _LANES))
    out = _normalize(vals.reshape(-1, seq))

    anchor = out[:batch].reshape(batch, seq, 1)
    positive = out[batch:2 * batch].reshape(batch, seq, 1)
    negative = out[2 * batch:].reshape(batch, num_neg, seq)
    return (anchor, positive, negative)
